# baseline (speedup 1.0000x reference)
import functools

import jax
import jax.numpy as jnp
from jax import lax
from jax.experimental import pallas as pl
from jax.experimental.pallas import tpu as pltpu
from jax.experimental.pallas import tpu_sc as plsc

NC = 2
NS = 16
CH = 128
F0 = 0.58

_mesh = plsc.VectorSubcoreMesh(core_axis_name="c", subcore_axis_name="s")


def _split_counts(e):
    nc_pair = -(-e // (NS * CH))
    nc0 = max(1, min(nc_pair - 1, round(nc_pair * F0)))
    nc1 = nc_pair - nc0
    return nc0, nc1


@functools.partial(jax.jit, static_argnames=("n_pad", "nc0", "nc1"))
def _sc_degree(dst0, dst1, *, n_pad, nc0, nc1):
    rpt = n_pad // NS
    ncm = max(nc0, nc1)

    @functools.partial(
        pl.kernel,
        mesh=_mesh,
        out_type=jax.ShapeDtypeStruct((NC, n_pad), jnp.float32),
        scratch_types=[
            pltpu.VMEM((ncm, CH), jnp.int32),
            pltpu.VMEM((CH,), jnp.float32),
            pltpu.VMEM((rpt,), jnp.float32),
            pltpu.VMEM_SHARED((n_pad,), jnp.float32),
        ],
    )
    def deg_kernel(dst0_hbm, dst1_hbm, out_hbm, didx, ones_v, zbuf, deg_sh):
        cid = lax.axis_index("c")
        sid = lax.axis_index("s")

        @pl.when(cid == 0)
        def _():
            pltpu.sync_copy(dst0_hbm.at[sid], didx.at[pl.ds(0, nc0)])

        @pl.when(cid == 1)
        def _():
            pltpu.sync_copy(dst1_hbm.at[sid], didx.at[pl.ds(0, nc1)])

        def init_ones(i, carry):
            ones_v[pl.ds(i * 16, 16)] = jnp.ones((16,), jnp.float32)
            return carry

        lax.fori_loop(0, CH // 16, init_ones, 0)

        def init_zeros(i, carry):
            zbuf[pl.ds(i * 16, 16)] = jnp.zeros((16,), jnp.float32)
            return carry

        lax.fori_loop(0, rpt // 16, init_zeros, 0)
        pltpu.sync_copy(zbuf, deg_sh.at[pl.ds(sid * rpt, rpt)])
        plsc.subcore_barrier()

        nloc = jnp.where(cid == 0, nc0, nc1)

        def body(j, carry):
            pltpu.sync_copy(ones_v, deg_sh.at[didx.at[j]], add=True)
            return carry

        lax.fori_loop(0, nloc, body, 0)
        plsc.subcore_barrier()
        pltpu.sync_copy(deg_sh.at[pl.ds(sid * rpt, rpt)],
                        out_hbm.at[cid, pl.ds(sid * rpt, rpt)])

    return deg_kernel(dst0, dst1)


@functools.partial(jax.jit, static_argnames=("n_pad", "nc0", "nc1", "h"))
def _sc_propagate(gp, src0, dst0, src1, dst1, *, n_pad, nc0, nc1, h):
    rpt = n_pad // NS
    ncm = max(nc0, nc1)

    @functools.partial(
        pl.kernel,
        mesh=_mesh,
        out_type=jax.ShapeDtypeStruct((NC, n_pad, h), jnp.float32),
        scratch_types=[
            pltpu.VMEM((ncm, CH), jnp.int32),
            pltpu.VMEM((ncm, CH), jnp.int32),
            pltpu.VMEM((CH, h), jnp.float32),
            pltpu.VMEM_SHARED((n_pad, h), jnp.float32),
            pltpu.SemaphoreType.DMA,
        ],
        compiler_params=pltpu.CompilerParams(use_tc_tiling_on_sc=False),
    )
    def prop_kernel(gp_hbm, src0_hbm, dst0_hbm, src1_hbm, dst1_hbm, out_hbm,
                    sidx, didx, rows, acc, gsem):
        cid = lax.axis_index("c")
        sid = lax.axis_index("s")

        @pl.when(cid == 0)
        def _():
            pltpu.sync_copy(src0_hbm.at[sid], sidx.at[pl.ds(0, nc0)])
            pltpu.sync_copy(dst0_hbm.at[sid], didx.at[pl.ds(0, nc0)])

        @pl.when(cid == 1)
        def _():
            pltpu.sync_copy(src1_hbm.at[sid], sidx.at[pl.ds(0, nc1)])
            pltpu.sync_copy(dst1_hbm.at[sid], didx.at[pl.ds(0, nc1)])

        def init_zeros(i, carry):
            for c in range(h // 16):
                rows[i, pl.ds(c * 16, 16)] = jnp.zeros((16,), jnp.float32)
            return carry

        lax.fori_loop(0, CH, init_zeros, 0)
        off = 0
        while off < rpt:
            w = min(CH, rpt - off)
            pltpu.sync_copy(rows.at[pl.ds(0, w)],
                            acc.at[pl.ds(sid * rpt + off, w)])
            off += w
        plsc.subcore_barrier()

        nloc = jnp.where(cid == 0, nc0, nc1)

        def body(j, carry):
            pltpu.async_copy(gp_hbm.at[sidx.at[j]], rows, gsem).wait()
            pltpu.sync_copy(rows, acc.at[didx.at[j]], add=True)
            return carry

        lax.fori_loop(0, nloc, body, 0)
        plsc.subcore_barrier()
        pltpu.sync_copy(acc.at[pl.ds(sid * rpt, rpt)],
                        out_hbm.at[cid, pl.ds(sid * rpt, rpt)])

    return prop_kernel(gp, src0, dst0, src1, dst1)


def _prep_body(deg_ref, x_ref, w_ref, dinv_ref, gp_ref):
    d = deg_ref[0] + deg_ref[1] + 1.0
    dinv = lax.rsqrt(d)
    dinv_ref[...] = dinv
    gp_ref[...] = jnp.dot(x_ref[...], w_ref[...],
                          preferred_element_type=jnp.float32) * dinv


def _layer_body(p_ref, gp_ref, dinv_ref, w_ref, out_ref):
    s = (p_ref[0] + p_ref[1] + gp_ref[...]) * dinv_ref[...]
    hcur = jnp.where(s > 0, s, 0.01 * s)
    out_ref[...] = jnp.dot(hcur, w_ref[...],
                           preferred_element_type=jnp.float32) * dinv_ref[...]


def _recon_body(pz_ref, gpz_ref, dinvz_ref, p_ref, gp_ref, dinv_ref,
                out_ref, z_ref):
    @pl.when(pl.program_id(0) == 0)
    def _():
        sz = (pz_ref[0] + pz_ref[1] + gpz_ref[...]) * dinvz_ref[...]
        z_ref[...] = jnp.where(sz > 0, sz, 0.01 * sz)

    s = (p_ref[0] + p_ref[1] + gp_ref[...]) * dinv_ref[...]
    hs = jnp.where(s > 0, s, 0.01 * s)
    zz = lax.dot_general(z_ref[...], hs,
                         (((1,), (1,)), ((), ())),
                         preferred_element_type=jnp.float32)
    out_ref[...] = jax.nn.sigmoid(zz)


def kernel(x, edge_index, last_batch_node, pos_edges, neg_edges, W1, W2, W3, W4):
    n, d = x.shape
    h = W1.shape[1]
    b = pos_edges.shape[0]
    e = edge_index.shape[1]

    n_pad = ((n + 1 + 1023) // 1024) * 1024
    nc0, nc1 = _split_counts(e)
    e0 = NS * nc0 * CH
    e_pad = NS * (nc0 + nc1) * CH

    pad = jnp.full((e_pad - e,), n, dtype=jnp.int32)
    src0 = edge_index[0, :e0].reshape(NS, nc0, CH)
    dst0 = edge_index[1, :e0].reshape(NS, nc0, CH)
    src1 = jnp.concatenate([edge_index[0, e0:], pad]).reshape(NS, nc1, CH)
    dst1 = jnp.concatenate([edge_index[1, e0:], pad]).reshape(NS, nc1, CH)

    x_pad = jnp.zeros((n_pad, d), x.dtype).at[:n].set(x)

    deg = _sc_degree(dst0, dst1, n_pad=n_pad,
                     nc0=nc0, nc1=nc1).reshape(NC, n_pad, 1)

    dinv, gp = pl.pallas_call(
        _prep_body,
        out_shape=[
            jax.ShapeDtypeStruct((n_pad, 1), jnp.float32),
            jax.ShapeDtypeStruct((n_pad, h), jnp.float32),
        ],
    )(deg, x_pad, W1)

    for w_mat in (W2, W3, W4):
        p = _sc_propagate(gp, src0, dst0, src1, dst1,
                          n_pad=n_pad, nc0=nc0, nc1=nc1, h=h)
        gp = pl.pallas_call(
            _layer_body,
            out_shape=jax.ShapeDtypeStruct((n_pad, h), jnp.float32),
        )(p, gp, dinv, w_mat)

    p = _sc_propagate(gp, src0, dst0, src1, dst1,
                      n_pad=n_pad, nc0=nc0, nc1=nc1, h=h)

    start = jnp.asarray(last_batch_node, jnp.int32) + 1 - b
    pz = lax.dynamic_slice_in_dim(p, start, b, axis=1)
    gpz = lax.dynamic_slice_in_dim(gp, start, b, axis=0)
    dinvz = lax.dynamic_slice_in_dim(dinv, start, b, axis=0)

    cb = 1024
    recon = pl.pallas_call(
        _recon_body,
        grid=(-(-n // cb),),
        in_specs=[
            pl.BlockSpec((NC, b, h), lambda j: (0, 0, 0)),
            pl.BlockSpec((b, h), lambda j: (0, 0)),
            pl.BlockSpec((b, 1), lambda j: (0, 0)),
            pl.BlockSpec((NC, cb, h), lambda j: (0, j, 0)),
            pl.BlockSpec((cb, h), lambda j: (j, 0)),
            pl.BlockSpec((cb, 1), lambda j: (j, 0)),
        ],
        out_specs=pl.BlockSpec((b, cb), lambda j: (0, j)),
        out_shape=jax.ShapeDtypeStruct((b, n), jnp.float32),
        scratch_shapes=[pltpu.VMEM((b, h), jnp.float32)],
    )(pz, gpz, dinvz, p, gp, dinv)

    return recon


# speedup vs baseline: 1.0188x; 1.0188x over previous
import functools

import jax
import jax.numpy as jnp
from jax import lax
from jax.experimental import pallas as pl
from jax.experimental.pallas import tpu as pltpu
from jax.experimental.pallas import tpu_sc as plsc

NC = 2
NS = 16
CH = 128
F0 = 0.556

_mesh = plsc.VectorSubcoreMesh(core_axis_name="c", subcore_axis_name="s")


def _split_counts(e):
    nc_pair = -(-e // (NS * CH))
    nc0 = max(1, min(nc_pair - 1, round(nc_pair * F0)))
    nc1 = nc_pair - nc0
    return nc0, nc1


@functools.partial(jax.jit, static_argnames=("n_pad", "nc0", "nc1"))
def _sc_degree(dst0, dst1, *, n_pad, nc0, nc1):
    rpt = n_pad // NS
    ncm = max(nc0, nc1)

    @functools.partial(
        pl.kernel,
        mesh=_mesh,
        out_type=jax.ShapeDtypeStruct((NC, n_pad), jnp.float32),
        scratch_types=[
            pltpu.VMEM((ncm, CH), jnp.int32),
            pltpu.VMEM((CH,), jnp.float32),
            pltpu.VMEM((rpt,), jnp.float32),
            pltpu.VMEM_SHARED((n_pad,), jnp.float32),
        ],
    )
    def deg_kernel(dst0_hbm, dst1_hbm, out_hbm, didx, ones_v, zbuf, deg_sh):
        cid = lax.axis_index("c")
        sid = lax.axis_index("s")

        @pl.when(cid == 0)
        def _():
            pltpu.sync_copy(dst0_hbm.at[sid], didx.at[pl.ds(0, nc0)])

        @pl.when(cid == 1)
        def _():
            pltpu.sync_copy(dst1_hbm.at[sid], didx.at[pl.ds(0, nc1)])

        def init_ones(i, carry):
            ones_v[pl.ds(i * 16, 16)] = jnp.ones((16,), jnp.float32)
            return carry

        lax.fori_loop(0, CH // 16, init_ones, 0)

        def init_zeros(i, carry):
            zbuf[pl.ds(i * 16, 16)] = jnp.zeros((16,), jnp.float32)
            return carry

        lax.fori_loop(0, rpt // 16, init_zeros, 0)
        pltpu.sync_copy(zbuf, deg_sh.at[pl.ds(sid * rpt, rpt)])
        plsc.subcore_barrier()

        nloc = jnp.where(cid == 0, nc0, nc1)

        def body(j, carry):
            pltpu.sync_copy(ones_v, deg_sh.at[didx.at[j]], add=True)
            return carry

        lax.fori_loop(0, nloc, body, 0)
        plsc.subcore_barrier()
        pltpu.sync_copy(deg_sh.at[pl.ds(sid * rpt, rpt)],
                        out_hbm.at[cid, pl.ds(sid * rpt, rpt)])

    return deg_kernel(dst0, dst1)


@functools.partial(jax.jit, static_argnames=("n_pad", "nc0", "nc1", "h"))
def _sc_propagate(gp, src0, dst0, src1, dst1, *, n_pad, nc0, nc1, h):
    rpt = n_pad // NS
    ncm = max(nc0, nc1)

    @functools.partial(
        pl.kernel,
        mesh=_mesh,
        out_type=jax.ShapeDtypeStruct((NC, n_pad, h), jnp.float32),
        scratch_types=[
            pltpu.VMEM((ncm, CH), jnp.int32),
            pltpu.VMEM((ncm, CH), jnp.int32),
            pltpu.VMEM((CH, h), jnp.float32),
            pltpu.VMEM_SHARED((n_pad, h), jnp.float32),
            pltpu.SemaphoreType.DMA,
        ],
        compiler_params=pltpu.CompilerParams(use_tc_tiling_on_sc=False),
    )
    def prop_kernel(gp_hbm, src0_hbm, dst0_hbm, src1_hbm, dst1_hbm, out_hbm,
                    sidx, didx, rows, acc, gsem):
        cid = lax.axis_index("c")
        sid = lax.axis_index("s")

        @pl.when(cid == 0)
        def _():
            pltpu.sync_copy(src0_hbm.at[sid], sidx.at[pl.ds(0, nc0)])
            pltpu.sync_copy(dst0_hbm.at[sid], didx.at[pl.ds(0, nc0)])

        @pl.when(cid == 1)
        def _():
            pltpu.sync_copy(src1_hbm.at[sid], sidx.at[pl.ds(0, nc1)])
            pltpu.sync_copy(dst1_hbm.at[sid], didx.at[pl.ds(0, nc1)])

        def init_zeros(i, carry):
            for c in range(h // 16):
                rows[i, pl.ds(c * 16, 16)] = jnp.zeros((16,), jnp.float32)
            return carry

        lax.fori_loop(0, CH, init_zeros, 0)
        off = 0
        while off < rpt:
            w = min(CH, rpt - off)
            pltpu.sync_copy(rows.at[pl.ds(0, w)],
                            acc.at[pl.ds(sid * rpt + off, w)])
            off += w
        plsc.subcore_barrier()

        nloc = jnp.where(cid == 0, nc0, nc1)

        def body(j, carry):
            pltpu.async_copy(gp_hbm.at[sidx.at[j]], rows, gsem).wait()
            pltpu.sync_copy(rows, acc.at[didx.at[j]], add=True)
            return carry

        lax.fori_loop(0, nloc, body, 0)
        plsc.subcore_barrier()
        pltpu.sync_copy(acc.at[pl.ds(sid * rpt, rpt)],
                        out_hbm.at[cid, pl.ds(sid * rpt, rpt)])

    return prop_kernel(gp, src0, dst0, src1, dst1)


def _prep_body(deg_ref, x_ref, w_ref, dinv_ref, gp_ref):
    d = deg_ref[0] + deg_ref[1] + 1.0
    dinv = lax.rsqrt(d)
    dinv_ref[...] = dinv
    gp_ref[...] = jnp.dot(x_ref[...], w_ref[...],
                          preferred_element_type=jnp.float32) * dinv


def _layer_body(p_ref, gp_ref, dinv_ref, w_ref, out_ref):
    s = (p_ref[0] + p_ref[1] + gp_ref[...]) * dinv_ref[...]
    hcur = jnp.where(s > 0, s, 0.01 * s)
    out_ref[...] = jnp.dot(hcur, w_ref[...],
                           preferred_element_type=jnp.float32) * dinv_ref[...]


def _recon_body(pz_ref, gpz_ref, dinvz_ref, p_ref, gp_ref, dinv_ref,
                out_ref, z_ref):
    @pl.when(pl.program_id(0) == 0)
    def _():
        sz = (pz_ref[0] + pz_ref[1] + gpz_ref[...]) * dinvz_ref[...]
        z_ref[...] = jnp.where(sz > 0, sz, 0.01 * sz)

    s = (p_ref[0] + p_ref[1] + gp_ref[...]) * dinv_ref[...]
    hs = jnp.where(s > 0, s, 0.01 * s)
    zz = lax.dot_general(z_ref[...], hs,
                         (((1,), (1,)), ((), ())),
                         preferred_element_type=jnp.float32)
    out_ref[...] = jax.nn.sigmoid(zz)


def kernel(x, edge_index, last_batch_node, pos_edges, neg_edges, W1, W2, W3, W4):
    n, d = x.shape
    h = W1.shape[1]
    b = pos_edges.shape[0]
    e = edge_index.shape[1]

    n_pad = ((n + 1 + 1023) // 1024) * 1024
    nc0, nc1 = _split_counts(e)
    e0 = NS * nc0 * CH
    e_pad = NS * (nc0 + nc1) * CH

    pad = jnp.full((e_pad - e,), n, dtype=jnp.int32)
    src0 = edge_index[0, :e0].reshape(NS, nc0, CH)
    dst0 = edge_index[1, :e0].reshape(NS, nc0, CH)
    src1 = jnp.concatenate([edge_index[0, e0:], pad]).reshape(NS, nc1, CH)
    dst1 = jnp.concatenate([edge_index[1, e0:], pad]).reshape(NS, nc1, CH)

    x_pad = jnp.zeros((n_pad, d), x.dtype).at[:n].set(x)

    deg = _sc_degree(dst0, dst1, n_pad=n_pad,
                     nc0=nc0, nc1=nc1).reshape(NC, n_pad, 1)

    dinv, gp = pl.pallas_call(
        _prep_body,
        out_shape=[
            jax.ShapeDtypeStruct((n_pad, 1), jnp.float32),
            jax.ShapeDtypeStruct((n_pad, h), jnp.float32),
        ],
    )(deg, x_pad, W1)

    for w_mat in (W2, W3, W4):
        p = _sc_propagate(gp, src0, dst0, src1, dst1,
                          n_pad=n_pad, nc0=nc0, nc1=nc1, h=h)
        gp = pl.pallas_call(
            _layer_body,
            out_shape=jax.ShapeDtypeStruct((n_pad, h), jnp.float32),
        )(p, gp, dinv, w_mat)

    p = _sc_propagate(gp, src0, dst0, src1, dst1,
                      n_pad=n_pad, nc0=nc0, nc1=nc1, h=h)

    start = jnp.asarray(last_batch_node, jnp.int32) + 1 - b
    pz = lax.dynamic_slice_in_dim(p, start, b, axis=1)
    gpz = lax.dynamic_slice_in_dim(gp, start, b, axis=0)
    dinvz = lax.dynamic_slice_in_dim(dinv, start, b, axis=0)

    cb = 1024
    recon = pl.pallas_call(
        _recon_body,
        grid=(-(-n // cb),),
        in_specs=[
            pl.BlockSpec((NC, b, h), lambda j: (0, 0, 0)),
            pl.BlockSpec((b, h), lambda j: (0, 0)),
            pl.BlockSpec((b, 1), lambda j: (0, 0)),
            pl.BlockSpec((NC, cb, h), lambda j: (0, j, 0)),
            pl.BlockSpec((cb, h), lambda j: (j, 0)),
            pl.BlockSpec((cb, 1), lambda j: (j, 0)),
        ],
        out_specs=pl.BlockSpec((b, cb), lambda j: (0, j)),
        out_shape=jax.ShapeDtypeStruct((b, n), jnp.float32),
        scratch_shapes=[pltpu.VMEM((b, h), jnp.float32)],
    )(pz, gpz, dinvz, p, gp, dinv)

    return recon
